# baseline (speedup 1.0000x reference)
import functools
from math import sqrt

import jax
import jax.numpy as jnp
from jax import lax
from jax.experimental import pallas as pl
from jax.experimental.pallas import tpu as pltpu
from jax.experimental.pallas import tpu_sc as plsc

D_MODEL = 64
SCALE = float(sqrt(D_MODEL))
LANES = 16

NUM_CORES = 2
NUM_SUBCORES = 16
NUM_WORKERS = NUM_CORES * NUM_SUBCORES

NRING = 2
OPITCH = 129


@functools.lru_cache(maxsize=None)
def _make_lookup(J: int, I: int, D: int):
    CB = 128
    assert I == CB * NUM_WORKERS and J % NRING == 0
    mesh = plsc.VectorSubcoreMesh(core_axis_name="c", subcore_axis_name="s")

    @functools.partial(
        pl.kernel,
        mesh=mesh,
        out_type=jax.ShapeDtypeStruct((J, D, I), jnp.float32),
        scratch_types=(
            [pltpu.VMEM((J, CB), jnp.int32)]
            + [pltpu.VMEM((CB, 2 * D), jnp.float32) for _ in range(NRING)]
            + [pltpu.VMEM((D, OPITCH), jnp.float32) for _ in range(NRING)]
            + [pltpu.SemaphoreType.DMA for _ in range(2 * NRING)]
        ),
        compiler_params=pltpu.CompilerParams(needs_layout_passes=False),
    )
    def lookup(x_hbm, tab_hbm, out_hbm, xcol_v, *rest):
        gbuf = rest[0:NRING]
        obuf = rest[NRING:2 * NRING]
        gsem = rest[2 * NRING:3 * NRING]
        ssem = rest[3 * NRING:4 * NRING]

        wid = lax.axis_index("s") * NUM_CORES + lax.axis_index("c")
        cbase = wid * CB
        pltpu.sync_copy(x_hbm.at[:, pl.ds(cbase, CB)], xcol_v)

        def gather(j, s):
            return pltpu.make_async_copy(
                tab_hbm.at[xcol_v.at[j]], gbuf[s], gsem[s]
            )

        def store(j, s):
            return pltpu.make_async_copy(
                obuf[s].at[:, pl.ds(0, CB)],
                out_hbm.at[j, :, pl.ds(cbase, CB)],
                ssem[s],
            )

        def transform(s):
            gb, ob = gbuf[s], obuf[s]
            dvecs = [
                lax.iota(jnp.int32, LANES) + (k * LANES)
                for k in range(D // LANES)
            ]

            @plsc.parallel_loop(0, CB, 1, unroll=4)
            def _(c):
                cvec = jnp.full((LANES,), 0, jnp.int32) + c
                for k in range(D // LANES):
                    v = gb[c, pl.ds(k * LANES, LANES)]
                    plsc.store_scatter(ob, [dvecs[k], cvec], v * SCALE)

        for s in range(NRING - 1):
            gather(s, s).start()

        def outer(o, carry):
            for s in range(NRING):
                j = o * NRING + s
                gather(j, s).wait()

                @pl.when(j >= NRING)
                def _():
                    store(j - NRING, s).wait()

                transform(s)
                store(j, s).start()
                sp = (s - 1) % NRING
                jn = j + NRING - 1

                @pl.when(jn < J)
                def _():
                    gather(jn, sp).start()

            return carry

        lax.fori_loop(0, J // NRING, outer, 0)

        for s in range(NRING):
            store(J - NRING + s, s).wait()

    return lookup


def kernel(x, table):
    J, I = x.shape[1], x.shape[0]
    xT = x.T.astype(jnp.int32)
    tab_p = jnp.pad(table, ((0, 0), (0, D_MODEL)))
    outP = _make_lookup(J, I, D_MODEL)(xT, tab_p)
    return outP.transpose(2, 0, 1)


# speedup vs baseline: 1.2419x; 1.2419x over previous
import functools
from math import sqrt

import jax
import jax.numpy as jnp
from jax import lax
from jax.experimental import pallas as pl
from jax.experimental.pallas import tpu as pltpu
from jax.experimental.pallas import tpu_sc as plsc

D_MODEL = 64
SCALE = float(sqrt(D_MODEL))
LANES = 16

NUM_CORES = 2
NUM_SUBCORES = 16
NUM_WORKERS = NUM_CORES * NUM_SUBCORES

NBUF = 4
CHUNK = 256


@functools.lru_cache(maxsize=None)
def _make_lookup(B: int, D: int):
    C = CHUNK
    assert B % (8 * NUM_WORKERS) == 0
    b_per_w = B // NUM_WORKERS
    assert b_per_w % (C * NBUF) == 0
    n_chunks = b_per_w // C
    n_outer = n_chunks // NBUF
    mesh = plsc.VectorSubcoreMesh(core_axis_name="c", subcore_axis_name="s")

    @functools.partial(
        pl.kernel,
        mesh=mesh,
        out_type=jax.ShapeDtypeStruct((B, D), jnp.float32),
        scratch_types=(
            [pltpu.VMEM((b_per_w,), jnp.int32)]
            + [pltpu.VMEM((C, D), jnp.float32) for _ in range(NBUF)]
            + [pltpu.SemaphoreType.DMA for _ in range(NBUF)]
        ),
        compiler_params=pltpu.CompilerParams(use_tc_tiling_on_sc=False),
    )
    def lookup(idx_hbm, table_hbm, out_hbm, idx_v, b0, b1, b2, b3, s0, s1, s2, s3):
        bufs = [b0, b1, b2, b3]
        sems = [s0, s1, s2, s3]
        wid = lax.axis_index("s") * NUM_CORES + lax.axis_index("c")
        base = wid * b_per_w
        pltpu.sync_copy(idx_hbm.at[pl.ds(base, b_per_w)], idx_v)

        def start_gather(g, b):
            pltpu.async_copy(
                table_hbm.at[idx_v.at[pl.ds(g * C, C)]], bufs[b], sems[b]
            )

        def wait_gather(g, b):
            pltpu.make_async_copy(
                table_hbm.at[idx_v.at[pl.ds(g * C, C)]], bufs[b], sems[b]
            ).wait()

        def start_store(g, b):
            pltpu.async_copy(bufs[b], out_hbm.at[pl.ds(base + g * C, C)], sems[b])

        def wait_store(b):
            pltpu.make_async_copy(
                bufs[b], out_hbm.at[pl.ds(base, C)], sems[b]
            ).wait()

        def scale_buf(b):
            buf = bufs[b]

            @plsc.parallel_loop(0, C, 1, unroll=8)
            def _(r):
                for j in range(D // LANES):
                    sl = pl.ds(j * LANES, LANES)
                    buf[r, sl] = buf[r, sl] * SCALE

        for b in range(NBUF - 1):
            start_gather(b, b)

        def outer(o, carry):
            for b in range(NBUF):
                g = o * NBUF + b
                wait_gather(g, b)
                scale_buf(b)
                start_store(g, b)
                bp = (b - 1) % NBUF
                g_next = g + NBUF - 1

                @pl.when(g_next < n_chunks)
                def _():
                    @pl.when(g > 0)
                    def _():
                        wait_store(bp)

                    start_gather(g_next, bp)

            return carry

        lax.fori_loop(0, n_outer, outer, 0)

        for b in range(NBUF):
            wait_store(b)

    return lookup


def kernel(x, table):
    B = x.shape[0] * x.shape[1]
    idx = x.reshape(-1).astype(jnp.int32)
    out = _make_lookup(B, D_MODEL)(idx, table)
    return out.reshape(x.shape[0], x.shape[1], D_MODEL)
